# baseline (speedup 1.0000x reference)
import functools

import jax
import jax.numpy as jnp
from jax import lax
from jax.experimental import pallas as pl
from jax.experimental.pallas import tpu as pltpu

F32 = jnp.float32
BF16 = jnp.bfloat16


def _fold_img(x):
    B = x.shape[0]
    x = x.reshape(B, 3, 28, 2, 4, 28, 2, 4)
    x = x.transpose(0, 3, 6, 2, 5, 1, 4, 7)
    return x.reshape(B, 4, 784, 48)


def _fold_img4(x):
    x = x.reshape(8, 4, 3, 28, 2, 4, 28, 2, 4)
    x = x.transpose(0, 4, 7, 3, 6, 1, 2, 5, 8)
    return x.reshape(8, 4, 784, 192)


def _fold_w1(w):
    cout = w.shape[0]
    w = w.reshape(cout, 3, 2, 4, 2, 4)
    w = w.transpose(2, 4, 1, 3, 5, 0)
    return w.reshape(4, 48, cout)


def _taps_w(w):
    o, i, kh, kw = w.shape
    return jnp.transpose(w, (2, 3, 1, 0)).reshape(kh * kw, i, o)


def _kron4(w):
    eye = jnp.eye(4, dtype=w.dtype)
    return jax.vmap(lambda m: jnp.kron(eye, m))(w)


def _blockdiag4(ws):
    return jax.vmap(jax.scipy.linalg.block_diag)(*ws)


def _shift(a, off):
    if off == 0:
        return a
    return jnp.pad(a[off:784, :], ((0, off), (0, 0)))


def _conv_stack_body(x_ref, w1_ref, w2_ref, w3_ref, b1_ref, b2_ref, b3_ref,
                     o_ref):
    cdt = x_ref.dtype
    xq = [x_ref[0, 784 * q:784 * (q + 1), :] for q in range(4)]
    y1 = []
    for ph in range(2):
        for pw in range(2):
            acc = jnp.zeros((784, 128), F32)
            for dh in range(2):
                for dw in range(2):
                    qh, oh = (ph + dh) % 2, (ph + dh) // 2
                    qw, ow = (pw + dw) % 2, (pw + dw) // 2
                    xs = _shift(xq[2 * qh + qw], 28 * oh + ow)
                    acc = acc + lax.dot(xs, w1_ref[2 * dh + dw],
                                        preferred_element_type=F32)
            y1.append(jnp.maximum(acc + b1_ref[:], 0.0).astype(cdt))
    acc = jnp.zeros((784, 256), F32)
    for kh in range(4):
        for kw in range(4):
            q = 2 * (kh % 2) + (kw % 2)
            off = 28 * (kh // 2) + (kw // 2)
            acc = acc + lax.dot(_shift(y1[q], off), w2_ref[4 * kh + kw],
                                preferred_element_type=F32)
    y2 = jnp.maximum(acc + b2_ref[:], 0.0).astype(cdt)
    acc = jnp.zeros((784, 256), F32)
    for di in range(3):
        for dj in range(3):
            acc = acc + lax.dot(_shift(y2, 28 * di + dj), w3_ref[3 * di + dj],
                                preferred_element_type=F32)
    y3 = acc + b3_ref[:]
    rows = lax.broadcasted_iota(jnp.int32, (784, 256), 0)
    valid = ((rows // 28) < 24) & ((rows % 28) < 24)
    y3 = jnp.where(valid, y3, 0.0)
    o_ref[0, 0, :] = jnp.sum(y3, axis=0) / 576.0


def _conv_stack_call(xf, w1, w2, w3, b1, b2, b3, grid):
    kin = xf.shape[-1]
    full = lambda a: pl.BlockSpec(a.shape, lambda i: (0,) * a.ndim)
    return pl.pallas_call(
        _conv_stack_body,
        grid=(grid,),
        in_specs=[
            pl.BlockSpec((1, 3136, kin), lambda i: (i, 0, 0)),
            full(w1), full(w2), full(w3), full(b1), full(b2), full(b3),
        ],
        out_specs=pl.BlockSpec((1, 1, 256), lambda i: (i, 0, 0)),
        out_shape=jax.ShapeDtypeStruct((grid, 1, 256), F32),
        compiler_params=pltpu.CompilerParams(
            dimension_semantics=("parallel",)),
    )(xf.reshape(grid, 3136, kin), w1, w2, w3, b1, b2, b3)[:, 0, :]


def _mlp3(x, w1, b1, w2, b2, w3, b3):
    x = jnp.maximum(lax.dot(x, w1, preferred_element_type=F32) + b1, 0.0)
    x = jnp.maximum(lax.dot(x, w2, preferred_element_type=F32) + b2, 0.0)
    return jnp.maximum(lax.dot(x, w3, preferred_element_type=F32) + b3, 0.0)


def _head_body(*refs):
    (fimg_ref, fsub_ref, h0_ref, l0_ref, at_ref, pb_ref, pbt_ref) = refs[:7]
    wl = [r[:] for r in refs[7:-2]]
    loss_ref, amse_ref = refs[-2], refs[-1]

    it = iter(wl)
    nxt = lambda n: [next(it) for _ in range(n)]

    f_img = _mlp3(fimg_ref[:], *nxt(6))
    h = h0_ref[:]
    w1, b1, w2, b2 = nxt(4)
    h = jnp.maximum(lax.dot(h, w1, preferred_element_type=F32) + b1, 0.0)
    h = jnp.maximum(lax.dot(h, w2, preferred_element_type=F32) + b2, 0.0)
    x = jnp.concatenate([f_img, h], axis=1)
    f_w = _mlp3(x, *nxt(6))
    w1, b1, w2, b2, w3, b3 = nxt(6)
    w = jnp.maximum(lax.dot(f_w, w1, preferred_element_type=F32) + b1, 0.0)
    w = jnp.maximum(lax.dot(w, w2, preferred_element_type=F32) + b2, 0.0)
    w_0 = jax.nn.sigmoid(lax.dot(w, w3, preferred_element_type=F32) + b3)

    pe = jax.nn.sigmoid(pb_ref[:])
    pet = jax.nn.sigmoid(pbt_ref[:])
    d = (jnp.sum(w_0 * w_0, axis=1, keepdims=True)
         + jnp.sum(pet * pet, axis=0, keepdims=True)
         - 2.0 * lax.dot(w_0, pet, preferred_element_type=F32))
    nd = -d
    m = jnp.max(nd, axis=1, keepdims=True)
    lse = jnp.log(jnp.sum(jnp.exp(nd - m), axis=1, keepdims=True)) + m
    p_pred = nd - lse
    xs = -l0_ref[:] / 1e-05
    m2 = jnp.max(xs, axis=1, keepdims=True)
    e = jnp.exp(xs - m2)
    p_true = e / jnp.sum(e, axis=1, keepdims=True)
    loss = -jnp.sum(p_true * p_pred) / 32.0
    loss_ref[:] = jnp.broadcast_to(loss, (8, 128))

    dmin = jnp.min(d, axis=1, keepdims=True)
    cols = lax.broadcasted_iota(jnp.int32, (32, 512), 1)
    idx = jnp.min(jnp.where(d == dmin, cols, 1 << 20), axis=1, keepdims=True)
    enc = (cols == idx).astype(F32)
    w_q = lax.dot(enc, pe, preferred_element_type=F32)

    fsub = fsub_ref[:]
    at = at_ref[:]
    num = jnp.zeros((32, 7), F32)
    den = jnp.zeros((32, 7), F32)
    for i in range(4):
        feat = _mlp3(fsub[:, 64 * i:64 * i + 64], *nxt(6))
        s1w, s1b, s2w, s2b, s3w, s3b = nxt(6)
        s = jnp.maximum(lax.dot(feat, s1w, preferred_element_type=F32) + s1b, 0.0)
        s = jnp.maximum(lax.dot(s, s2w, preferred_element_type=F32) + s2b, 0.0)
        s = lax.dot(s, s3w, preferred_element_type=F32) + s3b
        mu = s[:, :7]
        pre = s[:, 7:14]
        sd = jnp.maximum(pre, 0.0) + jnp.log1p(jnp.exp(-jnp.abs(pre))) + 0.001
        lv = jnp.log(sd)
        inv = w_q[:, i:i + 1] / (jnp.exp(lv) + 1e-06)
        num = num + inv * mu
        den = den + inv
    mean = num / (den + 1e-06)
    amse = jnp.mean((at[:, :7] - mean) ** 2, axis=1, keepdims=True)
    amse_ref[:] = jnp.broadcast_to(amse, (32, 128))


def kernel(x_0, x_t, h_0, l_0, a_t, params):
    p = params
    img = p['img']
    subs = p['sub']

    w1s = jnp.concatenate([_fold_w1(s['c1w']) for s in subs], axis=-1).astype(BF16)
    w2s = _blockdiag4([_taps_w(s['c2w']) for s in subs]).astype(BF16)
    w3s = _blockdiag4([_taps_w(s['c3w']) for s in subs]).astype(BF16)
    b1s = jnp.concatenate([s['c1b'] for s in subs])[None, :]
    b2s = jnp.concatenate([s['c2b'] for s in subs])[None, :]
    b3s = jnp.concatenate([s['c3b'] for s in subs])[None, :]

    w1i = _kron4(_fold_w1(img['c1w']))
    w2i = _kron4(_taps_w(img['c2w']))
    w3i = _kron4(_taps_w(img['c3w']))
    b1i = jnp.tile(img['c1b'], 4)[None, :]
    b2i = jnp.tile(img['c2b'], 4)[None, :]
    b3i = jnp.tile(img['c3b'], 4)[None, :]

    xtf = _fold_img(x_t.astype(BF16))
    x0f = _fold_img4(x_0)

    fsub = _conv_stack_call(xtf, w1s, w2s, w3s, b1s, b2s, b3s, grid=32)
    fimg4 = _conv_stack_call(x0f, w1i, w2i, w3i, b1i, b2i, b3i, grid=8)
    fimg = fimg4.reshape(8, 4, 64).reshape(32, 64)

    vec = lambda b: b[None, :]
    weights = []
    weights += [img['f1w'], vec(img['f1b']), img['f2w'], vec(img['f2b']),
                img['f3w'], vec(img['f3b'])]
    weights += [p['fs1w'], vec(p['fs1b']), p['fs2w'], vec(p['fs2b'])]
    weights += [p['fa1w'], vec(p['fa1b']), p['fa2w'], vec(p['fa2b']),
                p['fa3w'], vec(p['fa3b'])]
    weights += [p['fw1w'], vec(p['fw1b']), p['fw2w'], vec(p['fw2b']),
                p['fw3w'], vec(p['fw3b'])]
    for s in subs:
        weights += [s['f1w'], vec(s['f1b']), s['f2w'], vec(s['f2b']),
                    s['f3w'], vec(s['f3b'])]
        weights += [s['st1w'], vec(s['st1b']), s['st2w'], vec(s['st2b']),
                    s['st3w'], vec(s['st3b'])]

    ins = [fimg, fsub, h_0, l_0, a_t, p['playbook'], p['playbook'].T] + weights
    full = lambda a: pl.BlockSpec(a.shape, lambda: (0,) * a.ndim)
    loss2d, amse2d = pl.pallas_call(
        _head_body,
        in_specs=[full(a) for a in ins],
        out_specs=[pl.BlockSpec((8, 128), lambda: (0, 0)),
                   pl.BlockSpec((32, 128), lambda: (0, 0))],
        out_shape=[jax.ShapeDtypeStruct((8, 128), F32),
                   jax.ShapeDtypeStruct((32, 128), F32)],
    )(*ins)

    distill_loss = loss2d[0, 0]
    a_mse_loss = amse2d[:, 0]
    g_mse_loss = jnp.zeros((1,), F32)
    return distill_loss, a_mse_loss, g_mse_loss


# speedup vs baseline: 3.5676x; 3.5676x over previous
import functools

import jax
import jax.numpy as jnp
from jax import lax
from jax.experimental import pallas as pl
from jax.experimental.pallas import tpu as pltpu

F32 = jnp.float32
BF16 = jnp.bfloat16


def _fold_img(x):
    B = x.shape[0]
    x = x.reshape(B, 3, 28, 2, 4, 28, 2, 4)
    x = x.transpose(0, 3, 6, 2, 5, 1, 4, 7)
    return x.reshape(B, 4, 784, 48)


def _fold_img4(x):
    x = _fold_img(x).reshape(8, 4, 4, 784, 48)
    return x.transpose(0, 2, 3, 1, 4).reshape(8, 4, 784, 192)


def _fold_w1(w):
    cout = w.shape[0]
    w = w.reshape(cout, 3, 2, 4, 2, 4)
    w = w.transpose(2, 4, 1, 3, 5, 0)
    return w.reshape(4, 48, cout)


def _taps_w(w):
    o, i, kh, kw = w.shape
    return jnp.transpose(w, (2, 3, 1, 0)).reshape(kh * kw, i, o)


def _kron4(w):
    eye = jnp.eye(4, dtype=w.dtype)
    return jax.vmap(lambda m: jnp.kron(eye, m))(w)


def _blockdiag4(ws):
    return jax.vmap(jax.scipy.linalg.block_diag)(*ws)


def _shift(a, off):
    if off == 0:
        return a
    return jnp.pad(a[off:784, :], ((0, off), (0, 0)))


def _conv_stack_body(x_ref, w1_ref, w2_ref, w3_ref, b1_ref, b2_ref, b3_ref,
                     o_ref):
    cdt = w1_ref.dtype
    xq = [x_ref[0, 784 * q:784 * (q + 1), :].astype(cdt) for q in range(4)]
    y1 = []
    for ph in range(2):
        for pw in range(2):
            acc = jnp.zeros((784, 128), F32)
            for dh in range(2):
                for dw in range(2):
                    qh, oh = (ph + dh) % 2, (ph + dh) // 2
                    qw, ow = (pw + dw) % 2, (pw + dw) // 2
                    xs = _shift(xq[2 * qh + qw], 28 * oh + ow)
                    acc = acc + lax.dot(xs, w1_ref[2 * dh + dw],
                                        preferred_element_type=F32)
            y1.append(jnp.maximum(acc + b1_ref[:], 0.0).astype(cdt))
    acc = jnp.zeros((784, 256), F32)
    for kh in range(4):
        for kw in range(4):
            q = 2 * (kh % 2) + (kw % 2)
            off = 28 * (kh // 2) + (kw // 2)
            acc = acc + lax.dot(_shift(y1[q], off), w2_ref[4 * kh + kw],
                                preferred_element_type=F32)
    y2 = jnp.maximum(acc + b2_ref[:], 0.0).astype(cdt)
    acc = jnp.zeros((784, 256), F32)
    for di in range(3):
        for dj in range(3):
            acc = acc + lax.dot(_shift(y2, 28 * di + dj), w3_ref[3 * di + dj],
                                preferred_element_type=F32)
    y3 = acc + b3_ref[:]
    rows = lax.broadcasted_iota(jnp.int32, (784, 256), 0)
    valid = ((rows // 28) < 24) & ((rows % 28) < 24)
    y3 = jnp.where(valid, y3, 0.0)
    o_ref[0, 0, :] = jnp.sum(y3, axis=0) / 576.0


def _conv_stack_call(xf, w1, w2, w3, b1, b2, b3, grid):
    kin = xf.shape[-1]
    full = lambda a: pl.BlockSpec(a.shape, lambda i: (0,) * a.ndim)
    return pl.pallas_call(
        _conv_stack_body,
        grid=(grid,),
        in_specs=[
            pl.BlockSpec((1, 3136, kin), lambda i: (i, 0, 0)),
            full(w1), full(w2), full(w3), full(b1), full(b2), full(b3),
        ],
        out_specs=pl.BlockSpec((1, 1, 256), lambda i: (i, 0, 0)),
        out_shape=jax.ShapeDtypeStruct((grid, 1, 256), F32),
        compiler_params=pltpu.CompilerParams(
            dimension_semantics=("parallel",)),
    )(xf.reshape(grid, 3136, kin), w1, w2, w3, b1, b2, b3)[:, 0, :]


def _mlp3(x, w1, b1, w2, b2, w3, b3):
    x = jnp.maximum(lax.dot(x, w1, preferred_element_type=F32) + b1, 0.0)
    x = jnp.maximum(lax.dot(x, w2, preferred_element_type=F32) + b2, 0.0)
    return jnp.maximum(lax.dot(x, w3, preferred_element_type=F32) + b3, 0.0)


def _head_body(*refs):
    (fimg_ref, fsub_ref, h0_ref, l0_ref, at_ref, pb_ref, pbt_ref) = refs[:7]
    wl = [r[:] for r in refs[7:-2]]
    loss_ref, amse_ref = refs[-2], refs[-1]

    it = iter(wl)
    nxt = lambda n: [next(it) for _ in range(n)]

    f_img = _mlp3(fimg_ref[:], *nxt(6))
    h = h0_ref[:]
    w1, b1, w2, b2 = nxt(4)
    h = jnp.maximum(lax.dot(h, w1, preferred_element_type=F32) + b1, 0.0)
    h = jnp.maximum(lax.dot(h, w2, preferred_element_type=F32) + b2, 0.0)
    x = jnp.concatenate([f_img, h], axis=1)
    f_w = _mlp3(x, *nxt(6))
    w1, b1, w2, b2, w3, b3 = nxt(6)
    w = jnp.maximum(lax.dot(f_w, w1, preferred_element_type=F32) + b1, 0.0)
    w = jnp.maximum(lax.dot(w, w2, preferred_element_type=F32) + b2, 0.0)
    w_0 = jax.nn.sigmoid(lax.dot(w, w3, preferred_element_type=F32) + b3)

    pe = jax.nn.sigmoid(pb_ref[:])
    pet = jax.nn.sigmoid(pbt_ref[:])
    d = (jnp.sum(w_0 * w_0, axis=1, keepdims=True)
         + jnp.sum(pet * pet, axis=0, keepdims=True)
         - 2.0 * lax.dot(w_0, pet, preferred_element_type=F32))
    nd = -d
    m = jnp.max(nd, axis=1, keepdims=True)
    lse = jnp.log(jnp.sum(jnp.exp(nd - m), axis=1, keepdims=True)) + m
    p_pred = nd - lse
    xs = -l0_ref[:] / 1e-05
    m2 = jnp.max(xs, axis=1, keepdims=True)
    e = jnp.exp(xs - m2)
    p_true = e / jnp.sum(e, axis=1, keepdims=True)
    loss = -jnp.sum(p_true * p_pred) / 32.0
    loss_ref[:] = jnp.broadcast_to(loss, (8, 128))

    dmin = jnp.min(d, axis=1, keepdims=True)
    cols = lax.broadcasted_iota(jnp.int32, (32, 512), 1)
    idx = jnp.min(jnp.where(d == dmin, cols, 1 << 20), axis=1, keepdims=True)
    enc = (cols == idx).astype(F32)
    w_q = lax.dot(enc, pe, preferred_element_type=F32)

    fsub = fsub_ref[:]
    at = at_ref[:]
    num = jnp.zeros((32, 7), F32)
    den = jnp.zeros((32, 7), F32)
    for i in range(4):
        feat = _mlp3(fsub[:, 64 * i:64 * i + 64], *nxt(6))
        s1w, s1b, s2w, s2b, s3w, s3b = nxt(6)
        s = jnp.maximum(lax.dot(feat, s1w, preferred_element_type=F32) + s1b, 0.0)
        s = jnp.maximum(lax.dot(s, s2w, preferred_element_type=F32) + s2b, 0.0)
        s = lax.dot(s, s3w, preferred_element_type=F32) + s3b
        mu = s[:, :7]
        pre = s[:, 7:14]
        sd = jnp.maximum(pre, 0.0) + jnp.log1p(jnp.exp(-jnp.abs(pre))) + 0.001
        lv = jnp.log(sd)
        inv = w_q[:, i:i + 1] / (jnp.exp(lv) + 1e-06)
        num = num + inv * mu
        den = den + inv
    mean = num / (den + 1e-06)
    amse = jnp.mean((at[:, :7] - mean) ** 2, axis=1, keepdims=True)
    amse_ref[:] = jnp.broadcast_to(amse, (32, 128))


def kernel(x_0, x_t, h_0, l_0, a_t, params):
    p = params
    img = p['img']
    subs = p['sub']

    w1s = jnp.concatenate([_fold_w1(s['c1w']) for s in subs], axis=-1).astype(BF16)
    w2s = _blockdiag4([_taps_w(s['c2w']) for s in subs]).astype(BF16)
    w3s = _blockdiag4([_taps_w(s['c3w']) for s in subs]).astype(BF16)
    b1s = jnp.concatenate([s['c1b'] for s in subs])[None, :]
    b2s = jnp.concatenate([s['c2b'] for s in subs])[None, :]
    b3s = jnp.concatenate([s['c3b'] for s in subs])[None, :]

    w1i = _kron4(_fold_w1(img['c1w']))
    w2i = _kron4(_taps_w(img['c2w']))
    w3i = _kron4(_taps_w(img['c3w']))
    b1i = jnp.tile(img['c1b'], 4)[None, :]
    b2i = jnp.tile(img['c2b'], 4)[None, :]
    b3i = jnp.tile(img['c3b'], 4)[None, :]

    xtf = _fold_img(x_t)
    x0f = _fold_img4(x_0)

    fsub = _conv_stack_call(xtf, w1s, w2s, w3s, b1s, b2s, b3s, grid=32)
    fimg4 = _conv_stack_call(x0f, w1i, w2i, w3i, b1i, b2i, b3i, grid=8)
    fimg = fimg4.reshape(8, 4, 64).reshape(32, 64)

    vec = lambda b: b[None, :]
    weights = []
    weights += [img['f1w'], vec(img['f1b']), img['f2w'], vec(img['f2b']),
                img['f3w'], vec(img['f3b'])]
    weights += [p['fs1w'], vec(p['fs1b']), p['fs2w'], vec(p['fs2b'])]
    weights += [p['fa1w'], vec(p['fa1b']), p['fa2w'], vec(p['fa2b']),
                p['fa3w'], vec(p['fa3b'])]
    weights += [p['fw1w'], vec(p['fw1b']), p['fw2w'], vec(p['fw2b']),
                p['fw3w'], vec(p['fw3b'])]
    for s in subs:
        weights += [s['f1w'], vec(s['f1b']), s['f2w'], vec(s['f2b']),
                    s['f3w'], vec(s['f3b'])]
        weights += [s['st1w'], vec(s['st1b']), s['st2w'], vec(s['st2b']),
                    s['st3w'], vec(s['st3b'])]

    ins = [fimg, fsub, h_0, l_0, a_t, p['playbook'], p['playbook'].T] + weights
    full = lambda a: pl.BlockSpec(a.shape, lambda: (0,) * a.ndim)
    loss2d, amse2d = pl.pallas_call(
        _head_body,
        in_specs=[full(a) for a in ins],
        out_specs=[pl.BlockSpec((8, 128), lambda: (0, 0)),
                   pl.BlockSpec((32, 128), lambda: (0, 0))],
        out_shape=[jax.ShapeDtypeStruct((8, 128), F32),
                   jax.ShapeDtypeStruct((32, 128), F32)],
    )(*ins)

    distill_loss = loss2d[0, 0]
    a_mse_loss = amse2d[:, 0]
    g_mse_loss = jnp.zeros((1,), F32)
    return distill_loss, a_mse_loss, g_mse_loss
